# baseline (speedup 1.0000x reference)
import functools

import jax
import jax.numpy as jnp
from jax import lax
from jax.experimental import pallas as pl
from jax.experimental.pallas import tpu as pltpu
from jax.experimental.pallas import tpu_sc as plsc

_NC = 2
_NS = 16
_NW = _NC * _NS
_L = 16
_B = 128


def _round_up(a, b):
    return (a + b - 1) // b * b


def _mesh():
    return plsc.VectorSubcoreMesh(
        core_axis_name="c", subcore_axis_name="s",
        num_cores=_NC, num_subcores=_NS)


def _fill_rows(rows_v, val):
    def body(r, carry):
        rows_v[r] = jnp.full((_L,), val, jnp.float32)
        return carry
    lax.fori_loop(0, _B, body, 0)


def _sc_deg(dst2d, npad):
    chunks = dst2d.shape[0]
    cpw = chunks // _NW
    rpt = npad // _NS
    nco = rpt // _B

    @functools.partial(
        pl.kernel,
        out_type=jax.ShapeDtypeStruct((_NC, npad, _L), jnp.float32),
        mesh=_mesh(),
        compiler_params=pltpu.CompilerParams(use_tc_tiling_on_sc=False),
        scratch_types=[
            pltpu.VMEM((cpw, _B), jnp.int32),
            pltpu.VMEM((_B, _L), jnp.float32),
            pltpu.VMEM((npad // _NS, _L), jnp.float32),
            pltpu.VMEM_SHARED((npad, _L), jnp.float32),
            pltpu.SemaphoreType.DMA,
            pltpu.SemaphoreType.DMA,
        ],
    )
    def k(dst_hbm, out_hbm, dst_v, rows_v, stage_v, acc_sh, gsem, ssem):
        cid = lax.axis_index("c")
        sid = lax.axis_index("s")
        wid = cid * _NS + sid
        idx_cp = pltpu.async_copy(
            dst_hbm.at[pl.ds(wid * cpw, cpw)], dst_v, gsem)
        _fill_rows(rows_v, 1.0)
        for r in range(_B):
            stage_v[r] = jnp.zeros((_L,), jnp.float32)
        for j in range(nco):
            pltpu.async_copy(
                stage_v.at[pl.ds(0, _B)],
                acc_sh.at[pl.ds(sid * rpt + j * _B, _B)], ssem)
        idx_cp.wait()
        pltpu.make_async_copy(
            out_hbm.at[cid, pl.ds(0, rpt)], stage_v, ssem).wait()
        plsc.subcore_barrier()

        def body(j, carry):
            pltpu.async_copy(rows_v, acc_sh.at[dst_v.at[j]], ssem, add=True)
            return carry
        lax.fori_loop(0, cpw, body, 0)

        def drain(j, carry):
            pltpu.make_async_copy(
                out_hbm.at[cid, pl.ds(0, _B)], rows_v, ssem).wait()
            return carry
        lax.fori_loop(0, cpw, drain, 0)
        plsc.subcore_barrier()
        for j in range(nco):
            pltpu.async_copy(acc_sh.at[pl.ds(sid * rpt + j * _B, _B)],
                             stage_v.at[pl.ds(j * _B, _B)], gsem)
        pltpu.make_async_copy(
            out_hbm.at[cid, pl.ds(0, rpt)], stage_v, gsem).wait()
        for j in range(nco):
            pltpu.async_copy(stage_v.at[pl.ds(j * _B, _B)],
                             out_hbm.at[cid, pl.ds(sid * rpt + j * _B, _B)],
                             ssem)
        pltpu.make_async_copy(
            out_hbm.at[cid, pl.ds(0, rpt)], stage_v, ssem).wait()

    return k(dst2d)


def _sc_edge(table, src2d, dst2d, npad):
    chunks = dst2d.shape[0]
    cpw = chunks // _NW
    rpt = npad // _NS
    nco = rpt // _B

    @functools.partial(
        pl.kernel,
        out_type=jax.ShapeDtypeStruct((_NC, npad, _L), jnp.float32),
        mesh=_mesh(),
        compiler_params=pltpu.CompilerParams(use_tc_tiling_on_sc=False),
        scratch_types=[
            pltpu.VMEM((cpw, _B), jnp.int32),
            pltpu.VMEM((cpw, _B), jnp.int32),
            pltpu.VMEM((cpw * _B, _L), jnp.float32),
            pltpu.VMEM((_B, _L), jnp.float32),
            pltpu.VMEM_SHARED((npad, _L), jnp.float32),
            pltpu.SemaphoreType.DMA,
            pltpu.SemaphoreType.DMA,
            pltpu.SemaphoreType.DMA,
        ],
    )
    def k(table_hbm, src_hbm, dst_hbm, out_hbm, src_v, dst_v, rows_v,
          zbuf_v, acc_sh, gsem, hsem, ssem):
        cid = lax.axis_index("c")
        sid = lax.axis_index("s")
        wid = cid * _NS + sid
        half = cpw // 2
        src_cp = pltpu.async_copy(
            src_hbm.at[pl.ds(wid * cpw, cpw)], src_v, gsem)
        dst_cp = pltpu.async_copy(
            dst_hbm.at[pl.ds(wid * cpw, cpw)], dst_v, ssem)
        src_cp.wait()

        def fire_gather_a(j, carry):
            pltpu.async_copy(table_hbm.at[src_v.at[j]],
                             rows_v.at[pl.ds(j * _B, _B)], gsem)
            return carry
        lax.fori_loop(0, half, fire_gather_a, 0)

        def fire_gather_b(j, carry):
            pltpu.async_copy(table_hbm.at[src_v.at[j]],
                             rows_v.at[pl.ds(j * _B, _B)], hsem)
            return carry
        lax.fori_loop(half, cpw, fire_gather_b, 0)

        _fill_rows(zbuf_v, 0.0)
        for j in range(nco):
            pltpu.async_copy(zbuf_v,
                             acc_sh.at[pl.ds(sid * rpt + j * _B, _B)], ssem)
        dst_cp.wait()
        pltpu.make_async_copy(
            out_hbm.at[cid, pl.ds(0, nco * _B)],
            rows_v.at[pl.ds(0, nco * _B)], ssem).wait()
        plsc.subcore_barrier()
        pltpu.make_async_copy(
            out_hbm.at[cid, pl.ds(0, half * _B)],
            rows_v.at[pl.ds(0, half * _B)], gsem).wait()

        def fire_scatter_a(j, carry):
            pltpu.async_copy(rows_v.at[pl.ds(j * _B, _B)],
                             acc_sh.at[dst_v.at[j]], ssem, add=True)
            return carry
        lax.fori_loop(0, half, fire_scatter_a, 0)
        pltpu.make_async_copy(
            out_hbm.at[cid, pl.ds(0, (cpw - half) * _B)],
            rows_v.at[pl.ds(0, (cpw - half) * _B)], hsem).wait()

        def fire_scatter_b(j, carry):
            pltpu.async_copy(rows_v.at[pl.ds(j * _B, _B)],
                             acc_sh.at[dst_v.at[j]], ssem, add=True)
            return carry
        lax.fori_loop(half, cpw, fire_scatter_b, 0)
        pltpu.make_async_copy(
            out_hbm.at[cid, pl.ds(0, cpw * _B)], rows_v, ssem).wait()
        plsc.subcore_barrier()
        for j in range(nco):
            pltpu.async_copy(acc_sh.at[pl.ds(sid * rpt + j * _B, _B)],
                             rows_v.at[pl.ds(j * _B, _B)], gsem)
        pltpu.make_async_copy(
            out_hbm.at[cid, pl.ds(0, nco * _B)],
            rows_v.at[pl.ds(0, nco * _B)], gsem).wait()
        for j in range(nco):
            pltpu.async_copy(rows_v.at[pl.ds(j * _B, _B)],
                             out_hbm.at[cid, pl.ds(sid * rpt + j * _B, _B)],
                             ssem)
        pltpu.make_async_copy(
            out_hbm.at[cid, pl.ds(0, nco * _B)],
            rows_v.at[pl.ds(0, nco * _B)], ssem).wait()

    return k(table, src2d, dst2d)


def _tc_prep(x3, w1bd, dpk, n, npad):
    rk = n * _L // _B
    rc = npad * _L // _B
    d = x3.shape[2]
    g = _B // _L

    def mm_body(x_ref, w_ref, h_ref):
        h = jnp.dot(x_ref[:, 0, :], w_ref[0:d, :],
                    preferred_element_type=jnp.float32)
        for j in range(1, g):
            h = h + jnp.dot(x_ref[:, j, :], w_ref[j * d:(j + 1) * d, :],
                            preferred_element_type=jnp.float32)
        h_ref[...] = h

    hk = pl.pallas_call(
        mm_body,
        out_shape=jax.ShapeDtypeStruct((rk, _B), jnp.float32),
    )(x3, w1bd)

    def sc_body(h_ref, d_ref, hs_ref, di_ref):
        dd = d_ref[...]
        dinv = lax.rsqrt(1.0 + dd[:rk, :] + dd[rc:rc + rk, :])
        hs_ref[...] = h_ref[...] * dinv
        di_ref[...] = dinv

    return pl.pallas_call(
        sc_body,
        out_shape=[jax.ShapeDtypeStruct((rk, _B), jnp.float32),
                   jax.ShapeDtypeStruct((rk, _B), jnp.float32)],
    )(hk, dpk)


def _tc_mid(s1k, hs1k, dik, b1t, w2bd, n, npad):
    rk = n * _L // _B
    rc = npad * _L // _B

    def body(s_ref, h_ref, di_ref, b_ref, w_ref, o_ref):
        s = s_ref[...]
        di = di_ref[...]
        a = di * (s[:rk, :] + s[rc:rc + rk, :] + h_ref[...]) + b_ref[...]
        o1 = jnp.maximum(a, 0.0)
        o_ref[...] = di * jnp.dot(o1, w_ref[...],
                                  preferred_element_type=jnp.float32)

    return pl.pallas_call(
        body,
        out_shape=jax.ShapeDtypeStruct((rk, _B), jnp.float32),
    )(s1k, hs1k, dik, b1t, w2bd)


def _tc_final(s2k, hs2k, dik, b2t, n, npad):
    rk = n * _L // _B
    rc = npad * _L // _B

    g = _B // _L

    def body(s_ref, h_ref, di_ref, b_ref, m_ref, i_ref, o_ref):
        s = s_ref[...]
        z = di_ref[...] * (s[:rk, :] + s[rc:rc + rk, :] + h_ref[...]) \
            + b_ref[...]
        e = jnp.exp(z)
        sums = jnp.dot(e, m_ref[...], preferred_element_type=jnp.float32)
        outv = z - jnp.log(sums)
        eye = i_ref[...]
        for j in range(g):
            o_ref[:, j, :] = jnp.dot(outv, eye[:, j * _L:(j + 1) * _L],
                                     preferred_element_type=jnp.float32)

    ones_bd = jnp.kron(jnp.eye(g, dtype=jnp.float32),
                       jnp.ones((_L, _L), jnp.float32))
    eye128 = jnp.eye(_B, dtype=jnp.float32)
    return pl.pallas_call(
        body,
        out_shape=jax.ShapeDtypeStruct((rk, g, _L), jnp.float32),
    )(s2k, hs2k, dik, b2t, ones_bd, eye128)


def kernel(x, edge_index, W1, b1, W2, b2):
    n = x.shape[0]
    e = edge_index.shape[1]
    npad = _round_up(n + _B, _NS * _B)
    epad = _round_up(e, _NW * _B)
    pad = epad - e

    flat = edge_index.reshape(-1)
    src = flat[:e]
    dst = flat[e:]
    pad_iota = jnp.arange(pad, dtype=jnp.int32) % _B
    src2d = jnp.concatenate([src, pad_iota]).reshape(-1, _B)
    dst2d = jnp.concatenate([dst, n + pad_iota]).reshape(-1, _B)

    g = _B // _L
    b1t = jnp.tile(b1, g).reshape(1, _B)
    b2t = jnp.tile(b2, g).reshape(1, _B)
    w1bd = jnp.kron(jnp.eye(g, dtype=jnp.float32), W1)
    w2bd = jnp.kron(jnp.eye(g, dtype=jnp.float32), W2)
    x3 = x.reshape(n // g, g, x.shape[1])

    def pack(a):
        return a.reshape(-1, _B)

    degp = _sc_deg(dst2d, npad)
    hs1k, dik = _tc_prep(x3, w1bd, pack(degp), n, npad)
    s1 = _sc_edge(hs1k.reshape(n, _L), src2d, dst2d, npad)
    hs2k = _tc_mid(pack(s1), hs1k, dik, b1t, w2bd, n, npad)
    s2 = _sc_edge(hs2k.reshape(n, _L), src2d, dst2d, npad)
    return _tc_final(pack(s2), hs2k, dik, b2t, n, npad).reshape(n, _L)


# speedup vs baseline: 1.0087x; 1.0087x over previous
import functools

import jax
import jax.numpy as jnp
from jax import lax
from jax.experimental import pallas as pl
from jax.experimental.pallas import tpu as pltpu
from jax.experimental.pallas import tpu_sc as plsc

_NC = 2
_NS = 16
_NW = _NC * _NS
_L = 16
_B = 128


def _round_up(a, b):
    return (a + b - 1) // b * b


def _mesh():
    return plsc.VectorSubcoreMesh(
        core_axis_name="c", subcore_axis_name="s",
        num_cores=_NC, num_subcores=_NS)


def _fill_rows(rows_v, val):
    def body(r, carry):
        rows_v[r] = jnp.full((_L,), val, jnp.float32)
        return carry
    lax.fori_loop(0, _B, body, 0)


def _sc_deg(dst2d, npad):
    chunks = dst2d.shape[0]
    cpw = chunks // _NW
    rpt = npad // _NS
    nco = rpt // _B

    @functools.partial(
        pl.kernel,
        out_type=jax.ShapeDtypeStruct((_NC, npad, _L), jnp.float32),
        mesh=_mesh(),
        compiler_params=pltpu.CompilerParams(use_tc_tiling_on_sc=False),
        scratch_types=[
            pltpu.VMEM((cpw, _B), jnp.int32),
            pltpu.VMEM((_B, _L), jnp.float32),
            pltpu.VMEM((npad // _NS, _L), jnp.float32),
            pltpu.VMEM_SHARED((npad, _L), jnp.float32),
            pltpu.SemaphoreType.DMA,
            pltpu.SemaphoreType.DMA,
        ],
    )
    def k(dst_hbm, out_hbm, dst_v, rows_v, stage_v, acc_sh, gsem, ssem):
        cid = lax.axis_index("c")
        sid = lax.axis_index("s")
        wid = cid * _NS + sid
        idx_cp = pltpu.async_copy(
            dst_hbm.at[pl.ds(wid * cpw, cpw)], dst_v, gsem)
        _fill_rows(rows_v, 1.0)
        for r in range(_B):
            stage_v[r] = jnp.zeros((_L,), jnp.float32)
        for j in range(nco):
            pltpu.async_copy(
                stage_v.at[pl.ds(0, _B)],
                acc_sh.at[pl.ds(sid * rpt + j * _B, _B)], ssem)
        idx_cp.wait()
        pltpu.make_async_copy(
            out_hbm.at[cid, pl.ds(0, rpt)], stage_v, ssem).wait()
        plsc.subcore_barrier()

        def body(j, carry):
            pltpu.async_copy(rows_v, acc_sh.at[dst_v.at[j]], ssem, add=True)
            return carry
        lax.fori_loop(0, cpw, body, 0)

        def drain(j, carry):
            pltpu.make_async_copy(
                out_hbm.at[cid, pl.ds(0, _B)], rows_v, ssem).wait()
            return carry
        lax.fori_loop(0, cpw, drain, 0)
        plsc.subcore_barrier()
        for j in range(nco):
            pltpu.async_copy(acc_sh.at[pl.ds(sid * rpt + j * _B, _B)],
                             stage_v.at[pl.ds(j * _B, _B)], gsem)
        pltpu.make_async_copy(
            out_hbm.at[cid, pl.ds(0, rpt)], stage_v, gsem).wait()
        for j in range(nco):
            pltpu.async_copy(stage_v.at[pl.ds(j * _B, _B)],
                             out_hbm.at[cid, pl.ds(sid * rpt + j * _B, _B)],
                             ssem)
        pltpu.make_async_copy(
            out_hbm.at[cid, pl.ds(0, rpt)], stage_v, ssem).wait()

    return k(dst2d)


def _sc_edge(table, src2d, dst2d, npad):
    chunks = dst2d.shape[0]
    cpw = chunks // _NW
    rpt = npad // _NS
    nco = rpt // _B

    @functools.partial(
        pl.kernel,
        out_type=jax.ShapeDtypeStruct((_NC, npad, _L), jnp.float32),
        mesh=_mesh(),
        compiler_params=pltpu.CompilerParams(use_tc_tiling_on_sc=False),
        scratch_types=[
            pltpu.VMEM((cpw, _B), jnp.int32),
            pltpu.VMEM((cpw, _B), jnp.int32),
            pltpu.VMEM((cpw * _B, _L), jnp.float32),
            pltpu.VMEM((_B, _L), jnp.float32),
            pltpu.VMEM_SHARED((npad, _L), jnp.float32),
            pltpu.SemaphoreType.DMA,
            pltpu.SemaphoreType.DMA,
        ],
    )
    def k(table_hbm, src_hbm, dst_hbm, out_hbm, src_v, dst_v, rows_v,
          zbuf_v, acc_sh, gsem, ssem):
        cid = lax.axis_index("c")
        sid = lax.axis_index("s")
        wid = cid * _NS + sid
        src_cp = pltpu.async_copy(
            src_hbm.at[pl.ds(wid * cpw, cpw)], src_v, gsem)
        dst_cp = pltpu.async_copy(
            dst_hbm.at[pl.ds(wid * cpw, cpw)], dst_v, ssem)
        src_cp.wait()

        def fire_gather(j, carry):
            pltpu.async_copy(table_hbm.at[src_v.at[j]],
                             rows_v.at[pl.ds(j * _B, _B)], gsem)
            return carry
        lax.fori_loop(0, cpw, fire_gather, 0)

        _fill_rows(zbuf_v, 0.0)
        for j in range(nco):
            pltpu.async_copy(zbuf_v,
                             acc_sh.at[pl.ds(sid * rpt + j * _B, _B)], ssem)
        dst_cp.wait()
        pltpu.make_async_copy(
            out_hbm.at[cid, pl.ds(0, nco * _B)],
            rows_v.at[pl.ds(0, nco * _B)], ssem).wait()
        plsc.subcore_barrier()
        pltpu.make_async_copy(
            out_hbm.at[cid, pl.ds(0, cpw * _B)], rows_v, gsem).wait()

        def fire_scatter(j, carry):
            pltpu.async_copy(rows_v.at[pl.ds(j * _B, _B)],
                             acc_sh.at[dst_v.at[j]], ssem, add=True)
            return carry
        lax.fori_loop(0, cpw, fire_scatter, 0)
        pltpu.make_async_copy(
            out_hbm.at[cid, pl.ds(0, cpw * _B)], rows_v, ssem).wait()
        plsc.subcore_barrier()
        for j in range(nco):
            pltpu.async_copy(acc_sh.at[pl.ds(sid * rpt + j * _B, _B)],
                             rows_v.at[pl.ds(j * _B, _B)], gsem)
        pltpu.make_async_copy(
            out_hbm.at[cid, pl.ds(0, nco * _B)],
            rows_v.at[pl.ds(0, nco * _B)], gsem).wait()
        for j in range(nco):
            pltpu.async_copy(rows_v.at[pl.ds(j * _B, _B)],
                             out_hbm.at[cid, pl.ds(sid * rpt + j * _B, _B)],
                             ssem)
        pltpu.make_async_copy(
            out_hbm.at[cid, pl.ds(0, nco * _B)],
            rows_v.at[pl.ds(0, nco * _B)], ssem).wait()

    return k(table, src2d, dst2d)


def _tc_prep(x3, w1bd, dpk, n, npad):
    rk = n * _L // _B
    rc = npad * _L // _B
    d = x3.shape[2]
    g = _B // _L

    def mm_body(x_ref, w_ref, h_ref):
        h = jnp.dot(x_ref[:, 0, :], w_ref[0:d, :],
                    preferred_element_type=jnp.float32)
        for j in range(1, g):
            h = h + jnp.dot(x_ref[:, j, :], w_ref[j * d:(j + 1) * d, :],
                            preferred_element_type=jnp.float32)
        h_ref[...] = h

    hk = pl.pallas_call(
        mm_body,
        out_shape=jax.ShapeDtypeStruct((rk, _B), jnp.float32),
    )(x3, w1bd)

    def sc_body(h_ref, d_ref, hs_ref, di_ref):
        dd = d_ref[...]
        dinv = lax.rsqrt(1.0 + dd[:rk, :] + dd[rc:rc + rk, :])
        hs_ref[...] = h_ref[...] * dinv
        di_ref[...] = dinv

    return pl.pallas_call(
        sc_body,
        out_shape=[jax.ShapeDtypeStruct((rk, _B), jnp.float32),
                   jax.ShapeDtypeStruct((rk, _B), jnp.float32)],
    )(hk, dpk)


def _tc_mid(s1k, hs1k, dik, b1t, w2bd, n, npad):
    rk = n * _L // _B
    rc = npad * _L // _B

    def body(s_ref, h_ref, di_ref, b_ref, w_ref, o_ref):
        s = s_ref[...]
        di = di_ref[...]
        a = di * (s[:rk, :] + s[rc:rc + rk, :] + h_ref[...]) + b_ref[...]
        o1 = jnp.maximum(a, 0.0)
        o_ref[...] = di * jnp.dot(o1, w_ref[...],
                                  preferred_element_type=jnp.float32)

    return pl.pallas_call(
        body,
        out_shape=jax.ShapeDtypeStruct((rk, _B), jnp.float32),
    )(s1k, hs1k, dik, b1t, w2bd)


def _tc_final(s2k, hs2k, dik, b2t, n, npad):
    rk = n * _L // _B
    rc = npad * _L // _B

    g = _B // _L

    def body(s_ref, h_ref, di_ref, b_ref, m_ref, i_ref, o_ref):
        s = s_ref[...]
        z = di_ref[...] * (s[:rk, :] + s[rc:rc + rk, :] + h_ref[...]) \
            + b_ref[...]
        e = jnp.exp(z)
        sums = jnp.dot(e, m_ref[...], preferred_element_type=jnp.float32)
        outv = z - jnp.log(sums)
        eye = i_ref[...]
        for j in range(g):
            o_ref[:, j, :] = jnp.dot(outv, eye[:, j * _L:(j + 1) * _L],
                                     preferred_element_type=jnp.float32)

    ones_bd = jnp.kron(jnp.eye(g, dtype=jnp.float32),
                       jnp.ones((_L, _L), jnp.float32))
    eye128 = jnp.eye(_B, dtype=jnp.float32)
    return pl.pallas_call(
        body,
        out_shape=jax.ShapeDtypeStruct((rk, g, _L), jnp.float32),
    )(s2k, hs2k, dik, b2t, ones_bd, eye128)


def kernel(x, edge_index, W1, b1, W2, b2):
    n = x.shape[0]
    e = edge_index.shape[1]
    npad = _round_up(n + _B, _NS * _B)
    epad = _round_up(e, _NW * _B)
    pad = epad - e

    flat = edge_index.reshape(-1)
    src = flat[:e]
    dst = flat[e:]
    pad_iota = jnp.arange(pad, dtype=jnp.int32) % _B
    src2d = jnp.concatenate([src, pad_iota]).reshape(-1, _B)
    dst2d = jnp.concatenate([dst, n + pad_iota]).reshape(-1, _B)

    g = _B // _L
    b1t = jnp.tile(b1, g).reshape(1, _B)
    b2t = jnp.tile(b2, g).reshape(1, _B)
    w1bd = jnp.kron(jnp.eye(g, dtype=jnp.float32), W1)
    w2bd = jnp.kron(jnp.eye(g, dtype=jnp.float32), W2)
    x3 = x.reshape(n // g, g, x.shape[1])

    def pack(a):
        return a.reshape(-1, _B)

    degp = _sc_deg(dst2d, npad)
    hs1k, dik = _tc_prep(x3, w1bd, pack(degp), n, npad)
    s1 = _sc_edge(hs1k.reshape(n, _L), src2d, dst2d, npad)
    hs2k = _tc_mid(pack(s1), hs1k, dik, b1t, w2bd, n, npad)
    s2 = _sc_edge(hs2k.reshape(n, _L), src2d, dst2d, npad)
    return _tc_final(pack(s2), hs2k, dik, b2t, n, npad).reshape(n, _L)
